# baseline (speedup 1.0000x reference)
import functools

import jax
import jax.numpy as jnp
from jax import lax
from jax.experimental import pallas as pl
from jax.experimental.pallas import tpu as pltpu

B = 8
C = 384
CR = 24
K = 192
H = 224
W = 224
HW = H * W
ROWS = 392
LANES = 128
HCH = 56
NH = ROWS // HCH


def _mean_body(x_ref, o_ref):
    h = pl.program_id(1)
    s = jnp.sum(x_ref[...], axis=(2, 3)).reshape(1, 1, C)

    @pl.when(h == 0)
    def _():
        o_ref[...] = s

    @pl.when(h > 0)
    def _():
        o_ref[...] += s


def _logits_body(ys_ref, w1t_ref, b1_ref, w2t_ref, b2_ref, z_ref):
    y = ys_ref[...].reshape(B, C) / float(HW)
    h = jnp.dot(y, w1t_ref[...], preferred_element_type=jnp.float32)
    h = jnp.maximum(h + b1_ref[...], 0.0)
    z = jnp.dot(h, w2t_ref[...], preferred_element_type=jnp.float32)
    z_ref[...] = z + b2_ref[...]


def _select_body(a_ref, at_ref, idx_ref, w_ref):
    row_i = lax.broadcasted_iota(jnp.int32, (C, C), 0)
    col_i = lax.broadcasted_iota(jnp.int32, (C, C), 1)
    lte = (col_i <= row_i).astype(jnp.float32)
    ci = lax.broadcasted_iota(jnp.int32, (C, K), 0).astype(jnp.float32)
    ji = lax.broadcasted_iota(jnp.int32, (C, K), 1)
    for b in range(B):
        vrow = a_ref[pl.ds(b, 1), :]
        vcol = at_ref[:, pl.ds(b, 1)]
        vr = jnp.broadcast_to(vrow, (C, C))
        vc = jnp.broadcast_to(vcol, (C, C))
        gt = (vr > vc) | ((vr == vc) & (col_i < row_i))
        rank = jnp.sum(gt.astype(jnp.float32), axis=1, keepdims=True)
        maskb = rank < float(K)
        maskf = maskb.astype(jnp.float32)
        pos = jnp.dot(lte, maskf, preferred_element_type=jnp.float32) - 1.0
        posi = pos.astype(jnp.int32)
        oh = (jnp.broadcast_to(posi, (C, K)) == ji) & jnp.broadcast_to(maskb, (C, K))
        ohf = oh.astype(jnp.float32)
        idxf = jnp.sum(ci * ohf, axis=0, keepdims=True)
        wsel = jnp.sum(jnp.broadcast_to(vcol, (C, K)) * ohf, axis=0,
                       keepdims=True)
        idx_ref[pl.ds(b, 1), :] = idxf.astype(jnp.int32) + (b * C)
        w_ref[pl.ds(b, 1), :] = wsel


def _gather_body(idx_ref, w_ref, x_ref, o_ref):
    i = pl.program_id(0)
    o_ref[...] = x_ref[...] * w_ref[i]


def kernel(x, W1, b1, W2, b2):
    x3 = x.reshape(B * C, ROWS, LANES)

    ysum = pl.pallas_call(
        _mean_body,
        grid=(B, NH),
        in_specs=[pl.BlockSpec((1, C, HCH, LANES), lambda b, h: (b, 0, h, 0))],
        out_specs=pl.BlockSpec((1, 1, C), lambda b, h: (b, 0, 0)),
        out_shape=jax.ShapeDtypeStruct((B, 1, C), jnp.float32),
    )(x.reshape(B, C, ROWS, LANES))

    z = pl.pallas_call(
        _logits_body,
        out_shape=jax.ShapeDtypeStruct((B, C), jnp.float32),
    )(ysum, W1.T, b1.reshape(1, CR), W2.T, b2.reshape(1, C))

    a = jax.nn.sigmoid(z)

    idx_flat, wsel = pl.pallas_call(
        _select_body,
        out_shape=(jax.ShapeDtypeStruct((B, K), jnp.int32),
                   jax.ShapeDtypeStruct((B, K), jnp.float32)),
    )(a, a.T)

    out = pl.pallas_call(
        _gather_body,
        grid_spec=pltpu.PrefetchScalarGridSpec(
            num_scalar_prefetch=2,
            grid=(B * K,),
            in_specs=[pl.BlockSpec((1, ROWS, LANES),
                                   lambda i, idxr, wr: (idxr[i], 0, 0))],
            out_specs=pl.BlockSpec((1, ROWS, LANES),
                                   lambda i, idxr, wr: (i, 0, 0)),
        ),
        out_shape=jax.ShapeDtypeStruct((B * K, ROWS, LANES), jnp.float32),
    )(idx_flat.reshape(B * K), wsel.reshape(B * K), x3)

    return out.reshape(B, K, H, W)


# speedup vs baseline: 1.7931x; 1.7931x over previous
import functools

import jax
import jax.numpy as jnp
from jax import lax
from jax.experimental import pallas as pl
from jax.experimental.pallas import tpu as pltpu

B = 8
C = 384
CR = 24
K = 192
H = 224
W = 224
HW = H * W
HCH = 32
NH = H // HCH


def _mean_body(x_ref, o_ref):
    h = pl.program_id(1)
    s = jnp.sum(x_ref[...], axis=(2, 3)).reshape(1, 1, C)

    @pl.when(h == 0)
    def _():
        o_ref[...] = s

    @pl.when(h > 0)
    def _():
        o_ref[...] += s


def _logits_body(ys_ref, w1t_ref, b1_ref, w2t_ref, b2_ref, z_ref):
    y = ys_ref[...].reshape(B, C) / float(HW)
    h = jnp.dot(y, w1t_ref[...], preferred_element_type=jnp.float32)
    h = jnp.maximum(h + b1_ref[...], 0.0)
    z = jnp.dot(h, w2t_ref[...], preferred_element_type=jnp.float32)
    z_ref[...] = z + b2_ref[...]


def _select_body(a_ref, at_ref, idx_ref, w_ref):
    row_i = lax.broadcasted_iota(jnp.int32, (C, C), 0)
    col_i = lax.broadcasted_iota(jnp.int32, (C, C), 1)
    lte = (col_i <= row_i).astype(jnp.float32)
    ci = lax.broadcasted_iota(jnp.int32, (C, K), 0).astype(jnp.float32)
    ji = lax.broadcasted_iota(jnp.int32, (C, K), 1)
    for b in range(B):
        vrow = a_ref[pl.ds(b, 1), :]
        vcol = at_ref[:, pl.ds(b, 1)]
        vr = jnp.broadcast_to(vrow, (C, C))
        vc = jnp.broadcast_to(vcol, (C, C))
        gt = (vr > vc) | ((vr == vc) & (col_i < row_i))
        rank = jnp.sum(gt.astype(jnp.float32), axis=1, keepdims=True)
        maskb = rank < float(K)
        maskf = maskb.astype(jnp.float32)
        pos = jnp.dot(lte, maskf, preferred_element_type=jnp.float32) - 1.0
        posi = pos.astype(jnp.int32)
        oh = (jnp.broadcast_to(posi, (C, K)) == ji) & jnp.broadcast_to(maskb, (C, K))
        ohf = oh.astype(jnp.float32)
        idxf = jnp.sum(ci * ohf, axis=0, keepdims=True)
        wsel = jnp.sum(jnp.broadcast_to(vcol, (C, K)) * ohf, axis=0,
                       keepdims=True)
        idx_ref[pl.ds(b, 1), :] = idxf.astype(jnp.int32) + (b * C)
        w_ref[pl.ds(b, 1), :] = wsel


def _gather_body(idx_ref, w_ref, x_ref, o_ref):
    i = pl.program_id(0)
    o_ref[...] = x_ref[...] * w_ref[i]


def kernel(x, W1, b1, W2, b2):
    x3 = x.reshape(B * C, H, W)

    ysum = pl.pallas_call(
        _mean_body,
        grid=(B, NH),
        in_specs=[pl.BlockSpec((1, C, HCH, W), lambda b, h: (b, 0, h, 0))],
        out_specs=pl.BlockSpec((1, 1, C), lambda b, h: (b, 0, 0)),
        out_shape=jax.ShapeDtypeStruct((B, 1, C), jnp.float32),
    )(x)

    z = pl.pallas_call(
        _logits_body,
        out_shape=jax.ShapeDtypeStruct((B, C), jnp.float32),
    )(ysum, W1.T, b1.reshape(1, CR), W2.T, b2.reshape(1, C))

    a = jax.nn.sigmoid(z)

    idx_flat, wsel = pl.pallas_call(
        _select_body,
        out_shape=(jax.ShapeDtypeStruct((B, K), jnp.int32),
                   jax.ShapeDtypeStruct((B, K), jnp.float32)),
    )(a, a.T)

    out = pl.pallas_call(
        _gather_body,
        grid_spec=pltpu.PrefetchScalarGridSpec(
            num_scalar_prefetch=2,
            grid=(B * K,),
            in_specs=[pl.BlockSpec((1, H, W),
                                   lambda i, idxr, wr: (idxr[i], 0, 0))],
            out_specs=pl.BlockSpec((1, H, W),
                                   lambda i, idxr, wr: (i, 0, 0)),
        ),
        out_shape=jax.ShapeDtypeStruct((B * K, H, W), jnp.float32),
    )(idx_flat.reshape(B * K), wsel.reshape(B * K), x3)

    return out.reshape(B, K, H, W)


# speedup vs baseline: 5.4450x; 3.0366x over previous
import jax
import jax.numpy as jnp
from jax import lax
from jax.experimental import pallas as pl

B = 8
C = 384
CR = 24
K = 192
H = 224
W = 224
HW = H * W
MHCH = 28
NMH = H // MHCH
GHCH = 8
NGH = H // GHCH


def _mean_body(x_ref, o_ref):
    h = pl.program_id(1)
    s = jnp.sum(x_ref[...], axis=(1, 2)).reshape(1, 1, C)

    @pl.when(h == 0)
    def _():
        o_ref[...] = s

    @pl.when(h > 0)
    def _():
        o_ref[...] += s


def _logits_body(ys_ref, w1t_ref, b1_ref, w2t_ref, b2_ref, z_ref):
    y = ys_ref[...].reshape(B, C) / float(HW)
    h = jnp.dot(y, w1t_ref[...], preferred_element_type=jnp.float32)
    h = jnp.maximum(h + b1_ref[...], 0.0)
    z = jnp.dot(h, w2t_ref[...], preferred_element_type=jnp.float32)
    z_ref[...] = z + b2_ref[...]


def _select_body(a_ref, at_ref, g_ref):
    row_i = lax.broadcasted_iota(jnp.int32, (C, C), 0)
    col_i = lax.broadcasted_iota(jnp.int32, (C, C), 1)
    lte = (row_i <= col_i).astype(jnp.float32)
    jKC = lax.broadcasted_iota(jnp.int32, (K, C), 0)
    for b in range(B):
        vrow = a_ref[pl.ds(b, 1), :]
        vcol = at_ref[:, pl.ds(b, 1)]
        vr = jnp.broadcast_to(vrow, (C, C))
        vc = jnp.broadcast_to(vcol, (C, C))
        gt = (vc > vr) | ((vc == vr) & (row_i < col_i))
        rank = jnp.sum(gt.astype(jnp.float32), axis=0, keepdims=True)
        maskb = rank < float(K)
        pos = jnp.dot(maskb.astype(jnp.float32), lte,
                      preferred_element_type=jnp.float32)
        posi = pos.astype(jnp.int32) - 1
        oh = (jnp.broadcast_to(posi, (K, C)) == jKC) & jnp.broadcast_to(
            maskb, (K, C))
        gb = jnp.where(oh, jnp.broadcast_to(vrow, (K, C)), 0.0)
        g_ref[pl.ds(b, 1), :, :] = gb.reshape(1, K, C)


def _gather_body(g_ref, x_ref, o_ref):
    g = g_ref[0]
    for hh in range(GHCH):
        xrow = x_ref[0, hh]
        o = lax.dot_general(g, xrow, (((1,), (1,)), ((), ())),
                            preferred_element_type=jnp.float32)
        o_ref[0, :, hh, :] = o


def kernel(x, W1, b1, W2, b2):
    xt = jnp.transpose(x, (0, 2, 3, 1))

    ysum = pl.pallas_call(
        _mean_body,
        grid=(B, NMH),
        in_specs=[pl.BlockSpec((1, MHCH, W, C), lambda b, h: (b, h, 0, 0))],
        out_specs=pl.BlockSpec((1, 1, C), lambda b, h: (b, 0, 0)),
        out_shape=jax.ShapeDtypeStruct((B, 1, C), jnp.float32),
    )(xt)

    z = pl.pallas_call(
        _logits_body,
        out_shape=jax.ShapeDtypeStruct((B, C), jnp.float32),
    )(ysum, W1.T, b1.reshape(1, CR), W2.T, b2.reshape(1, C))

    a = jax.nn.sigmoid(z)

    G = pl.pallas_call(
        _select_body,
        out_shape=jax.ShapeDtypeStruct((B, K, C), jnp.float32),
    )(a, a.T)

    out = pl.pallas_call(
        _gather_body,
        grid=(B, NGH),
        in_specs=[pl.BlockSpec((1, K, C), lambda b, h: (b, 0, 0)),
                  pl.BlockSpec((1, GHCH, W, C), lambda b, h: (b, h, 0, 0))],
        out_specs=pl.BlockSpec((1, K, GHCH, W), lambda b, h: (b, 0, h, 0)),
        out_shape=jax.ShapeDtypeStruct((B, K, H, W), jnp.float32),
    )(G, xt)

    return out


# speedup vs baseline: 6.2017x; 1.1390x over previous
import jax
import jax.numpy as jnp
from jax import lax
from jax.experimental import pallas as pl

B = 8
C = 384
CR = 24
K = 192
H = 224
W = 224
HW = H * W
MHCH = 56
NMH = H // MHCH
GHCH = 32
NGH = H // GHCH


def _mean_body(x_ref, o_ref):
    h = pl.program_id(1)
    s = jnp.sum(x_ref[...], axis=(1, 2)).reshape(1, 1, C)

    @pl.when(h == 0)
    def _():
        o_ref[...] = s

    @pl.when(h > 0)
    def _():
        o_ref[...] += s


def _logits_body(ys_ref, w1t_ref, b1_ref, w2t_ref, b2_ref, z_ref):
    y = ys_ref[...].reshape(B, C) / float(HW)
    h = jnp.dot(y, w1t_ref[...], preferred_element_type=jnp.float32)
    h = jnp.maximum(h + b1_ref[...], 0.0)
    z = jnp.dot(h, w2t_ref[...], preferred_element_type=jnp.float32)
    z_ref[...] = z + b2_ref[...]


def _select_body(a_ref, at_ref, g_ref):
    row_i = lax.broadcasted_iota(jnp.int32, (C, C), 0)
    col_i = lax.broadcasted_iota(jnp.int32, (C, C), 1)
    lte = (row_i <= col_i).astype(jnp.float32)
    jKC = lax.broadcasted_iota(jnp.int32, (K, C), 0)
    for b in range(B):
        vrow = a_ref[pl.ds(b, 1), :]
        vcol = at_ref[:, pl.ds(b, 1)]
        vr = jnp.broadcast_to(vrow, (C, C))
        vc = jnp.broadcast_to(vcol, (C, C))
        gt = (vc > vr) | ((vc == vr) & (row_i < col_i))
        rank = jnp.sum(gt.astype(jnp.float32), axis=0, keepdims=True)
        maskb = rank < float(K)
        pos = jnp.dot(maskb.astype(jnp.float32), lte,
                      preferred_element_type=jnp.float32)
        posi = pos.astype(jnp.int32) - 1
        oh = (jnp.broadcast_to(posi, (K, C)) == jKC) & jnp.broadcast_to(
            maskb, (K, C))
        gb = jnp.where(oh, jnp.broadcast_to(vrow, (K, C)), 0.0)
        g_ref[pl.ds(b, 1), :, :] = gb.reshape(1, K, C)


def _gather_body(g_ref, x_ref, o_ref):
    g = g_ref[0]
    for hh in range(GHCH):
        xrow = x_ref[0, hh]
        o = lax.dot_general(g, xrow, (((1,), (1,)), ((), ())),
                            preferred_element_type=jnp.float32)
        o_ref[0, :, hh, :] = o


def kernel(x, W1, b1, W2, b2):
    xt = jnp.transpose(x, (0, 2, 3, 1))

    ysum = pl.pallas_call(
        _mean_body,
        grid=(B, NMH),
        in_specs=[pl.BlockSpec((1, MHCH, W, C), lambda b, h: (b, h, 0, 0))],
        out_specs=pl.BlockSpec((1, 1, C), lambda b, h: (b, 0, 0)),
        out_shape=jax.ShapeDtypeStruct((B, 1, C), jnp.float32),
    )(xt)

    z = pl.pallas_call(
        _logits_body,
        out_shape=jax.ShapeDtypeStruct((B, C), jnp.float32),
    )(ysum, W1.T, b1.reshape(1, CR), W2.T, b2.reshape(1, C))

    a = jax.nn.sigmoid(z)

    G = pl.pallas_call(
        _select_body,
        out_shape=jax.ShapeDtypeStruct((B, K, C), jnp.float32),
    )(a, a.T)

    out = pl.pallas_call(
        _gather_body,
        grid=(B, NGH),
        in_specs=[pl.BlockSpec((1, K, C), lambda b, h: (b, 0, 0)),
                  pl.BlockSpec((1, GHCH, W, C), lambda b, h: (b, h, 0, 0))],
        out_specs=pl.BlockSpec((1, K, GHCH, W), lambda b, h: (b, 0, h, 0)),
        out_shape=jax.ShapeDtypeStruct((B, K, H, W), jnp.float32),
    )(G, xt)

    return out
